# baseline (speedup 1.0000x reference)
import functools

import jax
import jax.numpy as jnp
from jax import lax
from jax.experimental import pallas as pl
from jax.experimental.pallas import tpu as pltpu
from jax.experimental.pallas import tpu_sc as plsc

_B, _C, _T, _H, _W = 8, 3, 32, 224, 224
_NF = 4
_STRIDE = 8
_NBLK = _B * _C * _NF
_NC = 2
_NS = 16
_NW = _NC * _NS
_BLK_PER_W = _NBLK // _NW


def _sc_frame_gather(x3):
    mesh = plsc.VectorSubcoreMesh(core_axis_name="c", subcore_axis_name="s")

    @functools.partial(
        pl.kernel,
        mesh=mesh,
        out_type=jax.ShapeDtypeStruct((_NBLK, _H, _W), jnp.float32),
    )
    def k(x_hbm, out_hbm):
        wid = lax.axis_index("s") * _NC + lax.axis_index("c")
        for kk in range(_BLK_PER_W):
            g = wid * _BLK_PER_W + kk
            bc = g // _NF
            j = g % _NF
            src = bc * _T + _STRIDE * j
            pltpu.sync_copy(x_hbm.at[src], out_hbm.at[g])

    return k(x3)


def kernel(x):
    x3 = x.reshape(_B * _C * _T, _H, _W)
    out = _sc_frame_gather(x3)
    return out.reshape(_B, _C, _NF, _H, _W)


# speedup vs baseline: 19.6024x; 19.6024x over previous
import functools

import jax
import jax.numpy as jnp
from jax import lax
from jax.experimental import pallas as pl
from jax.experimental.pallas import tpu as pltpu
from jax.experimental.pallas import tpu_sc as plsc

_B, _C, _T, _H, _W = 8, 3, 32, 224, 224
_NF = 4
_STRIDE = 8
_NBLK = _B * _C * _NF
_NC = 2
_NS = 16
_NW = _NC * _NS
_BLK_PER_W = _NBLK // _NW


def _sc_frame_gather(x3):
    mesh = plsc.VectorSubcoreMesh(core_axis_name="c", subcore_axis_name="s")

    @functools.partial(
        pl.kernel,
        mesh=mesh,
        out_type=jax.ShapeDtypeStruct((_NBLK, _H, _W), jnp.float32),
        scratch_types=[
            pltpu.VMEM((_H, _W), jnp.float32),
            pltpu.VMEM((_H, _W), jnp.float32),
            pltpu.SemaphoreType.DMA,
            pltpu.SemaphoreType.DMA,
            pltpu.SemaphoreType.DMA,
            pltpu.SemaphoreType.DMA,
        ],
    )
    def k(x_hbm, out_hbm, buf0, buf1, si0, si1, so0, so1):
        wid = lax.axis_index("s") * _NC + lax.axis_index("c")
        bufs = (buf0, buf1)
        sis = (si0, si1)
        sos = (so0, so1)

        def offs(kk):
            g = wid * _BLK_PER_W + kk
            bc = g // _NF
            j = g % _NF
            return bc * _T + _STRIDE * j, g

        gathers = [None, None]
        scatters = [None, None]
        for kk in range(_BLK_PER_W):
            s = kk % 2
            src, _ = offs(kk)
            if scatters[s] is not None:
                scatters[s].wait()
            gathers[s] = pltpu.async_copy(x_hbm.at[src], bufs[s], sis[s])
            if kk >= 1:
                p = (kk - 1) % 2
                gathers[p].wait()
                _, pdst = offs(kk - 1)
                scatters[p] = pltpu.async_copy(
                    bufs[p], out_hbm.at[pdst], sos[p]
                )
        last = (_BLK_PER_W - 1) % 2
        gathers[last].wait()
        _, ldst = offs(_BLK_PER_W - 1)
        scatters[last] = pltpu.async_copy(
            bufs[last], out_hbm.at[ldst], sos[last]
        )
        for s in range(2):
            if scatters[s] is not None:
                scatters[s].wait()

    return k(x3)


def kernel(x):
    x3 = x.reshape(_B * _C * _T, _H, _W)
    out = _sc_frame_gather(x3)
    return out.reshape(_B, _C, _NF, _H, _W)
